# baseline (speedup 1.0000x reference)
import functools

import jax
import jax.numpy as jnp
from jax import lax
from jax.experimental import pallas as pl
from jax.experimental.pallas import tpu as pltpu
from jax.experimental.pallas import tpu_sc as plsc

_E = 64
_C = 2
_NC = 2
_NS = 16
_NW = _NC * _NS
_G = 128
_L = 16
_BLKV = 32768


def _project_kernel(x_ref, w_ref, b_ref, out0_ref, out1_ref):
    res = lax.dot_general(
        w_ref[...], x_ref[...], (((1,), (0,)), ((), ())),
        preferred_element_type=jnp.float32)
    res = res + b_ref[...]
    out0_ref[...] = res[0]
    out1_ref[...] = res[1]


def _project(tt, Wq, bq):
    vocab = tt.shape[1]
    grid = (vocab + _BLKV - 1) // _BLKV
    return pl.pallas_call(
        _project_kernel,
        grid=(grid,),
        in_specs=[
            pl.BlockSpec((_E, _BLKV), lambda i: (0, i)),
            pl.BlockSpec((_C, _E), lambda i: (0, 0)),
            pl.BlockSpec((_C, 1), lambda i: (0, 0)),
        ],
        out_specs=[
            pl.BlockSpec((_BLKV,), lambda i: (i,)),
            pl.BlockSpec((_BLKV,), lambda i: (i,)),
        ],
        out_shape=[
            jax.ShapeDtypeStruct((vocab,), jnp.float32),
            jax.ShapeDtypeStruct((vocab,), jnp.float32),
        ],
    )(tt, Wq, bq)


def _make_gather(n_tokens):
    tok_per_w = n_tokens // _NW
    n_groups = tok_per_w // _G
    mesh = plsc.VectorSubcoreMesh(core_axis_name="c", subcore_axis_name="s")

    @functools.partial(
        pl.kernel,
        out_type=jax.ShapeDtypeStruct((_NW, _C * tok_per_w), jnp.float32),
        mesh=mesh,
        compiler_params=pltpu.CompilerParams(
            needs_layout_passes=False, use_tc_tiling_on_sc=False),
        scratch_types=[
            pltpu.VMEM((tok_per_w,), jnp.int32),
            pltpu.VMEM((tok_per_w,), jnp.float32),
            pltpu.VMEM((tok_per_w,), jnp.float32),
            pltpu.VMEM((_C * tok_per_w,), jnp.float32),
            pltpu.SemaphoreType.DMA,
        ],
    )
    def k(p0_hbm, p1_hbm, idx_hbm, out_hbm, idx_v, s0_v, s1_v, s01_v, sem):
        wid = lax.axis_index("s") * _NC + lax.axis_index("c")
        pltpu.sync_copy(idx_hbm.at[wid], idx_v)
        handles = []
        for j in range(n_groups):
            sl = pl.ds(j * _G, _G)
            handles.append(
                pltpu.async_copy(p0_hbm.at[idx_v.at[sl]], s0_v.at[sl], sem))
            handles.append(
                pltpu.async_copy(p1_hbm.at[idx_v.at[sl]], s1_v.at[sl], sem))
        for h in handles:
            h.wait()

        lanes2 = lax.iota(jnp.int32, _L) * _C

        def mix_body(g, _):
            sl = pl.ds(g * _L, _L)
            pos = jnp.full((_L,), g * _L * _C, jnp.int32) + lanes2
            plsc.store_scatter(s01_v, [pos], s0_v[sl])
            plsc.store_scatter(s01_v, [pos + 1], s1_v[sl])
            return 0

        lax.fori_loop(0, tok_per_w // _L, mix_body, 0)
        pltpu.sync_copy(s01_v, out_hbm.at[wid])

    return k


@jax.jit
def kernel(sentence1, emb_table, W, b):
    batch, seq = sentence1.shape
    n_tokens = batch * seq
    tt = emb_table.T
    p0, p1 = _project(tt, W, b.reshape(_C, 1))
    idx = sentence1.reshape(_NW, n_tokens // _NW)
    out = _make_gather(n_tokens)(p0, p1, idx)
    return out.reshape(batch, seq, _C)


# speedup vs baseline: 2.2248x; 2.2248x over previous
import functools

import jax
import jax.numpy as jnp
from jax import lax
from jax.experimental import pallas as pl
from jax.experimental.pallas import tpu as pltpu
from jax.experimental.pallas import tpu_sc as plsc

_E = 64
_C = 2
_NC = 2
_NS = 16
_NW = _NC * _NS
_G = 128
_BLKV = 32768


def _project_kernel(x_ref, w_ref, b_ref, out0_ref, out1_ref):
    res = lax.dot_general(
        w_ref[...], x_ref[...], (((1,), (0,)), ((), ())),
        preferred_element_type=jnp.float32)
    res = res + b_ref[...]
    out0_ref[...] = res[0]
    out1_ref[...] = res[1]


def _project(tt, Wq, bq):
    vocab = tt.shape[1]
    grid = (vocab + _BLKV - 1) // _BLKV
    return pl.pallas_call(
        _project_kernel,
        grid=(grid,),
        in_specs=[
            pl.BlockSpec((_E, _BLKV), lambda i: (0, i)),
            pl.BlockSpec((_C, _E), lambda i: (0, 0)),
            pl.BlockSpec((_C, 1), lambda i: (0, 0)),
        ],
        out_specs=[
            pl.BlockSpec((_BLKV,), lambda i: (i,)),
            pl.BlockSpec((_BLKV,), lambda i: (i,)),
        ],
        out_shape=[
            jax.ShapeDtypeStruct((vocab,), jnp.float32),
            jax.ShapeDtypeStruct((vocab,), jnp.float32),
        ],
    )(tt, Wq, bq)


def _make_gather(n_tokens):
    tok_per_w = n_tokens // _NW
    n_groups = tok_per_w // _G
    mesh = plsc.VectorSubcoreMesh(core_axis_name="c", subcore_axis_name="s")

    @functools.partial(
        pl.kernel,
        out_type=jax.ShapeDtypeStruct((_C, _NW, tok_per_w), jnp.float32),
        mesh=mesh,
        compiler_params=pltpu.CompilerParams(
            needs_layout_passes=False, use_tc_tiling_on_sc=False),
        scratch_types=[
            pltpu.VMEM((tok_per_w,), jnp.int32),
            pltpu.VMEM((tok_per_w,), jnp.float32),
            pltpu.VMEM((tok_per_w,), jnp.float32),
            pltpu.SemaphoreType.DMA,
        ],
    )
    def k(p0_hbm, p1_hbm, idx_hbm, out_hbm, idx_v, s0_v, s1_v, sem):
        wid = lax.axis_index("s") * _NC + lax.axis_index("c")
        pltpu.sync_copy(idx_hbm.at[wid], idx_v)
        handles = []
        for j in range(n_groups):
            sl = pl.ds(j * _G, _G)
            handles.append(
                pltpu.async_copy(p0_hbm.at[idx_v.at[sl]], s0_v.at[sl], sem))
            handles.append(
                pltpu.async_copy(p1_hbm.at[idx_v.at[sl]], s1_v.at[sl], sem))
        for h in handles:
            h.wait()
        pltpu.sync_copy(s0_v, out_hbm.at[0, wid])
        pltpu.sync_copy(s1_v, out_hbm.at[1, wid])

    return k


@jax.jit
def kernel(sentence1, emb_table, W, b):
    batch, seq = sentence1.shape
    n_tokens = batch * seq
    tt = emb_table.T
    p0, p1 = _project(tt, W, b.reshape(_C, 1))
    idx = sentence1.reshape(_NW, n_tokens // _NW)
    out = _make_gather(n_tokens)(p0, p1, idx)
    return jnp.moveaxis(out.reshape(_C, batch, seq), 0, 2)
